# baseline (speedup 1.0000x reference)
import functools

import numpy as np
import jax
import jax.numpy as jnp
from jax import lax
from jax.experimental import pallas as pl
from jax.experimental.pallas import tpu as pltpu
from jax.experimental.pallas import tpu_sc as plsc

N = 10000
E = 320000
D_FEAT = 128
HIDDEN = 32
HEADS = 8
HC = HEADS * HIDDEN
N_GRAPHS = 64
OUT = 128
EPS = 1e-5
INV_SQRT_C = float(1.0 / np.sqrt(HIDDEN))

RB = 400
NRB = N // RB
B = 72
TILES = 16
EPT = E // TILES
NBLK = 277
TAIL = EPT - NBLK * B
NPT_MAIN = 640
NPT_LAST = N - 15 * NPT_MAIN

_f32 = jnp.float32


_sc_mesh = plsc.VectorSubcoreMesh(core_axis_name="c", subcore_axis_name="s")


@functools.partial(
    pl.kernel,
    out_type=[jax.ShapeDtypeStruct((N, 144), _f32),
              jax.ShapeDtypeStruct((N, 144), _f32)],
    mesh=_sc_mesh,
    compiler_params=pltpu.CompilerParams(needs_layout_passes=False,
                                         use_tc_tiling_on_sc=False),
    scratch_types=[
        pltpu.VMEM((B, 128), _f32),
        pltpu.VMEM((B, 128), _f32),
        pltpu.VMEM((B, 128), _f32),
        pltpu.VMEM((B, 144), _f32),
        pltpu.VMEM((B,), jnp.int32),
        pltpu.VMEM((B,), jnp.int32),
        pltpu.VMEM((TAIL,), jnp.int32),
        pltpu.VMEM((TAIL,), jnp.int32),
        pltpu.VMEM_SHARED((N, 144), _f32),
        pltpu.SemaphoreType.DMA,
        pltpu.SemaphoreType.DMA,
        pltpu.SemaphoreType.DMA,
    ],
)
def _sc_attn(q0, q1, k0, k1, v0, v1, dst, src, z144,
             acc0, acc1,
             qd, kt, vt, msg, dstv, srcv, dstv_t, srcv_t, acc_sh,
             sem1, sem2, sem3):
    c = lax.axis_index("c")
    s = lax.axis_index("s")
    iota16 = lax.iota(jnp.int32, 16)

    def run(qc, kc, vc, accc):
        ebase = s * EPT
        rbase = s * NPT_MAIN

        def for_chunks(rows, fn):
            nfull = rows // B
            rem = rows - nfull * B
            for j in range(nfull):
                fn(j * B, B)
            if rem:
                fn(nfull * B, rem)

        def zero_chunk(off, sz):
            pltpu.sync_copy(z144.at[pl.ds(off, sz)], msg.at[pl.ds(0, sz)])
            pltpu.sync_copy(msg.at[pl.ds(0, sz)], acc_sh.at[pl.ds(off, sz)])

        @pl.when(s < 15)
        def _():
            for_chunks(NPT_MAIN, lambda o, n: zero_chunk(rbase + o, n))

        @pl.when(s == 15)
        def _():
            for_chunks(NPT_LAST, lambda o, n: zero_chunk(15 * NPT_MAIN + o, n))

        plsc.subcore_barrier()

        def do_block(e0, nB, dv, sv):
            pltpu.sync_copy(dst.at[pl.ds(e0, nB)], dv)
            pltpu.sync_copy(src.at[pl.ds(e0, nB)], sv)
            cp1 = pltpu.async_copy(qc.at[dv], qd.at[pl.ds(0, nB)], sem1)
            cp2 = pltpu.async_copy(kc.at[sv], kt.at[pl.ds(0, nB)], sem2)
            cp3 = pltpu.async_copy(vc.at[sv], vt.at[pl.ds(0, nB)], sem3)
            cp1.wait()
            cp2.wait()
            cp3.wait()

            half = nB // 2

            def one_edge(e):
                exrow = jnp.zeros((16,), _f32)
                for h in range(4):
                    c0 = h * 32
                    qa = qd[e, pl.ds(c0, 16)]
                    qb = qd[e, pl.ds(c0 + 16, 16)]
                    ka = kt[e, pl.ds(c0, 16)]
                    kb = kt[e, pl.ds(c0 + 16, 16)]
                    sc = jnp.sum(qa * ka + qb * kb)
                    ev = jnp.exp(jnp.full((16,), sc, _f32))
                    msg[e, pl.ds(c0, 16)] = vt[e, pl.ds(c0, 16)] * ev
                    msg[e, pl.ds(c0 + 16, 16)] = vt[e, pl.ds(c0 + 16, 16)] * ev
                    exrow = jnp.where(iota16 == h, ev, exrow)
                msg[e, pl.ds(128, 16)] = jnp.where(iota16 < 4, exrow,
                                                   jnp.zeros((16,), _f32))

            def edge_body(i, carry2):
                one_edge(i)
                one_edge(i + half)
                return carry2

            lax.fori_loop(0, half, edge_body, 0)

            pltpu.sync_copy(msg.at[pl.ds(0, nB)], acc_sh.at[dv], add=True)

        def block(j, carry):
            do_block(ebase + j * B, B, dstv, srcv)
            return carry

        lax.fori_loop(0, NBLK, block, 0)
        do_block(ebase + NBLK * B, TAIL, dstv_t, srcv_t)
        plsc.subcore_barrier()

        def dump_chunk(off, sz):
            pltpu.sync_copy(acc_sh.at[pl.ds(off, sz)], msg.at[pl.ds(0, sz)])
            pltpu.sync_copy(msg.at[pl.ds(0, sz)], accc.at[pl.ds(off, sz)])

        @pl.when(s < 15)
        def _():
            for_chunks(NPT_MAIN, lambda o, n: dump_chunk(rbase + o, n))

        @pl.when(s == 15)
        def _():
            for_chunks(NPT_LAST, lambda o, n: dump_chunk(15 * NPT_MAIN + o, n))

    @pl.when(c == 0)
    def _():
        run(q0, k0, v0, acc0)

    @pl.when(c == 1)
    def _():
        run(q1, k1, v1, acc1)


def _dot(a, b):
    return jnp.dot(a, b, preferred_element_type=_f32)


def _qkvs_body0(x, wi, bi, wq, bq, wk, bk, wv, bv, ws, bs,
                q0, q1, k0, k1, v0, v1, skip):
    h = _dot(x[...], wi[...]) + bi[...]
    _qkvs_common(h, wq, bq, wk, bk, wv, bv, ws, bs,
                 q0, q1, k0, k1, v0, v1, skip)


def _qkvs_body(pre, s1, s2, gamma, beta, wq, bq, wk, bk, wv, bv, ws, bs,
               q0, q1, k0, k1, v0, v1, skip):
    mean = s1[...] * (1.0 / N)
    var = s2[...] * (1.0 / N) - mean * mean
    inv = lax.rsqrt(var + EPS)
    h = jnp.maximum((pre[...] - mean) * inv * gamma[...] + beta[...], 0.0)
    _qkvs_common(h, wq, bq, wk, bk, wv, bv, ws, bs,
                 q0, q1, k0, k1, v0, v1, skip)


def _qkvs_common(h, wq, bq, wk, bk, wv, bv, ws, bs,
                 q0, q1, k0, k1, v0, v1, skip):
    q = (_dot(h, wq[...]) + bq[...]) * INV_SQRT_C
    q0[...] = q[:, :128]
    q1[...] = q[:, 128:]
    k = _dot(h, wk[...]) + bk[...]
    k0[...] = k[:, :128]
    k1[...] = k[:, 128:]
    v = _dot(h, wv[...]) + bv[...]
    v0[...] = v[:, :128]
    v1[...] = v[:, 128:]
    skip[...] = _dot(h, ws[...]) + bs[...]


def _gate_body(skip, a0, a1, wb, pre, s1, s2):
    i = pl.program_id(0)
    sk = skip[...]
    acc0 = a0[...]
    acc1 = a1[...]
    den8 = jnp.concatenate([acc0[:, 128:132], acc1[:, 128:132]], axis=1)
    den_e = jnp.broadcast_to(den8[:, :, None], (RB, 8, 32)).reshape(RB, HC)
    ag = jnp.concatenate([acc0[:, :128], acc1[:, :128]], axis=1)
    ag = ag / (den_e + 1e-16)
    w_s = wb[0:1, :] + wb[2:3, :]
    w_a = wb[1:2, :] - wb[2:3, :]
    gl = (jnp.sum(sk * w_s, axis=1, keepdims=True)
          + jnp.sum(ag * w_a, axis=1, keepdims=True))
    g = jax.nn.sigmoid(gl)
    p = g * sk + (1.0 - g) * ag
    pre[...] = p

    @pl.when(i == 0)
    def _():
        s1[...] = jnp.zeros_like(s1)
        s2[...] = jnp.zeros_like(s2)

    s1[...] += jnp.sum(p, axis=0, keepdims=True)
    s2[...] += jnp.sum(p * p, axis=0, keepdims=True)


def _final_body(pre, s1, s2, gamma, beta, bat, wf, bf, out, acc, cnt):
    i = pl.program_id(0)
    mean = s1[...] * (1.0 / N)
    var = s2[...] * (1.0 / N) - mean * mean
    inv = lax.rsqrt(var + EPS)
    h = jnp.maximum((pre[...] - mean) * inv * gamma[...] + beta[...], 0.0)
    b = bat[...].reshape(1, RB)
    oh = (lax.broadcasted_iota(jnp.int32, (N_GRAPHS, RB), 0) == b).astype(_f32)

    @pl.when(i == 0)
    def _():
        acc[...] = jnp.zeros_like(acc)
        cnt[...] = jnp.zeros_like(cnt)

    acc[...] += lax.dot_general(oh, h, (((1,), (0,)), ((), ())),
                                preferred_element_type=_f32)
    cnt[...] += jnp.sum(oh, axis=1, keepdims=True)

    @pl.when(i == NRB - 1)
    def _():
        pooled = acc[...] / jnp.maximum(cnt[...], 1.0)
        out[...] = _dot(pooled, wf[...]) + bf[...]


def _row_spec(w):
    return pl.BlockSpec((RB, w), lambda i: (i, 0))


def _full_spec(shape):
    nd = len(shape)
    return pl.BlockSpec(shape, lambda i: (0,) * nd)


_QKVS_OUTS = (
    [jax.ShapeDtypeStruct((N, 128), _f32)] * 6
    + [jax.ShapeDtypeStruct((N, HC), _f32)]
)
_QKVS_OUT_SPECS = [_row_spec(128)] * 6 + [_row_spec(HC)]


def _qkvs0_call(x, wi, bi, wq, bq, wk, bk, wv, bv, ws, bs):
    return pl.pallas_call(
        _qkvs_body0,
        grid=(NRB,),
        in_specs=[_row_spec(D_FEAT),
                  _full_spec((D_FEAT, HIDDEN)), _full_spec((1, HIDDEN))]
                 + [_full_spec((HIDDEN, HC)), _full_spec((1, HC))] * 4,
        out_specs=_QKVS_OUT_SPECS,
        out_shape=_QKVS_OUTS,
    )(x, wi, bi, wq, bq, wk, bk, wv, bv, ws, bs)


def _qkvs_call(pre, s1, s2, gamma, beta, wq, bq, wk, bk, wv, bv, ws, bs):
    return pl.pallas_call(
        _qkvs_body,
        grid=(NRB,),
        in_specs=[_row_spec(HC)] + [_full_spec((1, HC))] * 4
                 + [_full_spec((HC, HC)), _full_spec((1, HC))] * 4,
        out_specs=_QKVS_OUT_SPECS,
        out_shape=_QKVS_OUTS,
    )(pre, s1, s2, gamma, beta, wq, bq, wk, bk, wv, bv, ws, bs)


def _gate_call(skip, a0, a1, wb3):
    return pl.pallas_call(
        _gate_body,
        grid=(NRB,),
        in_specs=[_row_spec(HC), _row_spec(144), _row_spec(144),
                  _full_spec((3, HC))],
        out_specs=[_row_spec(HC), _full_spec((1, HC)), _full_spec((1, HC))],
        out_shape=[jax.ShapeDtypeStruct((N, HC), _f32),
                   jax.ShapeDtypeStruct((1, HC), _f32),
                   jax.ShapeDtypeStruct((1, HC), _f32)],
    )(skip, a0, a1, wb3)


def _final_call(pre, s1, s2, gamma, beta, b3, wf, bf):
    return pl.pallas_call(
        _final_body,
        grid=(NRB,),
        in_specs=[_row_spec(HC)] + [_full_spec((1, HC))] * 4
                 + [pl.BlockSpec((1, 1, RB), lambda i: (i, 0, 0)),
                    _full_spec((HC, OUT)), _full_spec((1, OUT))],
        out_specs=[_full_spec((N_GRAPHS, OUT))],
        out_shape=[jax.ShapeDtypeStruct((N_GRAPHS, OUT), _f32)],
        scratch_shapes=[pltpu.VMEM((N_GRAPHS, HC), _f32),
                        pltpu.VMEM((N_GRAPHS, 1), _f32)],
    )(pre, s1, s2, gamma, beta, b3, wf, bf)[0]


def kernel(x, edge_index, batch, params):
    src = edge_index[0]
    dst = edge_index[1]
    z144 = jnp.zeros((N, 144), _f32)
    r1 = lambda a: a.reshape(1, -1)

    layers = params['layers']
    lp = layers[0]
    q0, q1, k0, k1, v0, v1, skip = _qkvs0_call(
        x, params['W_init'], r1(params['b_init']),
        lp['Wq'], r1(lp['bq']), lp['Wk'], r1(lp['bk']),
        lp['Wv'], r1(lp['bv']), lp['Wskip'], r1(lp['bskip']))
    acc0, acc1 = _sc_attn(q0, q1, k0, k1, v0, v1, dst, src, z144)
    pre, s1, s2 = _gate_call(skip, acc0, acc1, lp['Wbeta'].reshape(3, HC))

    for li in range(1, 4):
        prev = layers[li - 1]
        lp = layers[li]
        q0, q1, k0, k1, v0, v1, skip = _qkvs_call(
            pre, s1, s2, r1(prev['bn_gamma']), r1(prev['bn_beta']),
            lp['Wq'], r1(lp['bq']), lp['Wk'], r1(lp['bk']),
            lp['Wv'], r1(lp['bv']), lp['Wskip'], r1(lp['bskip']))
        acc0, acc1 = _sc_attn(q0, q1, k0, k1, v0, v1, dst, src, z144)
        pre, s1, s2 = _gate_call(skip, acc0, acc1, lp['Wbeta'].reshape(3, HC))

    lp = layers[3]
    b3 = batch.reshape(NRB, 1, RB)
    return _final_call(pre, s1, s2, r1(lp['bn_gamma']), r1(lp['bn_beta']),
                       b3, params['W_final'], r1(params['b_final']))


# speedup vs baseline: 1.0335x; 1.0335x over previous
import functools

import numpy as np
import jax
import jax.numpy as jnp
from jax import lax
from jax.experimental import pallas as pl
from jax.experimental.pallas import tpu as pltpu
from jax.experimental.pallas import tpu_sc as plsc

N = 10000
E = 320000
D_FEAT = 128
HIDDEN = 32
HEADS = 8
HC = HEADS * HIDDEN
N_GRAPHS = 64
OUT = 128
EPS = 1e-5
INV_SQRT_C = float(1.0 / np.sqrt(HIDDEN))

RB = 400
NRB = N // RB
B = 80
TILES = 16
EPT = E // TILES
NBLK = EPT // B
NPT_MAIN = 640
NPT_LAST = N - 15 * NPT_MAIN

_f32 = jnp.float32


_sc_mesh = plsc.VectorSubcoreMesh(core_axis_name="c", subcore_axis_name="s")


@functools.partial(
    pl.kernel,
    out_type=[jax.ShapeDtypeStruct((N, 144), _f32),
              jax.ShapeDtypeStruct((N, 144), _f32)],
    mesh=_sc_mesh,
    compiler_params=pltpu.CompilerParams(needs_layout_passes=False,
                                         use_tc_tiling_on_sc=False),
    scratch_types=[
        pltpu.VMEM((B, 128), _f32),
        pltpu.VMEM((B, 144), _f32),
        pltpu.VMEM((B, 128), _f32),
        pltpu.VMEM((B,), jnp.int32),
        pltpu.VMEM((B,), jnp.int32),
        pltpu.VMEM_SHARED((N, 144), _f32),
        pltpu.SemaphoreType.DMA,
        pltpu.SemaphoreType.DMA,
        pltpu.SemaphoreType.DMA,
    ],
)
def _sc_attn(q0, q1, k0, k1, v0, v1, dst, src, z144,
             acc0, acc1,
             qd, kt, vt, dstv, srcv, acc_sh,
             sem1, sem2, sem3):
    c = lax.axis_index("c")
    s = lax.axis_index("s")
    iota16 = lax.iota(jnp.int32, 16)

    def run(qc, kc, vc, accc):
        ebase = s * EPT
        rbase = s * NPT_MAIN

        def for_chunks(rows, fn):
            nfull = rows // B
            rem = rows - nfull * B
            for j in range(nfull):
                fn(j * B, B)
            if rem:
                fn(nfull * B, rem)

        def zero_chunk(off, sz):
            pltpu.sync_copy(z144.at[pl.ds(off, sz)], kt.at[pl.ds(0, sz)])
            pltpu.sync_copy(kt.at[pl.ds(0, sz)], acc_sh.at[pl.ds(off, sz)])

        @pl.when(s < 15)
        def _():
            for_chunks(NPT_MAIN, lambda o, n: zero_chunk(rbase + o, n))

        @pl.when(s == 15)
        def _():
            for_chunks(NPT_LAST, lambda o, n: zero_chunk(15 * NPT_MAIN + o, n))

        plsc.subcore_barrier()

        def block(j, carry):
            e0 = ebase + j * B
            pltpu.sync_copy(dst.at[pl.ds(e0, B)], dstv)
            pltpu.sync_copy(src.at[pl.ds(e0, B)], srcv)
            cp1 = pltpu.async_copy(qc.at[dstv], qd, sem1)
            cp2 = pltpu.async_copy(kc.at[srcv], kt, sem2)
            cp3 = pltpu.async_copy(vc.at[srcv], vt, sem3)
            cp1.wait()
            cp2.wait()
            cp3.wait()

            def edge_body(e, carry2):
                exrow = jnp.zeros((16,), _f32)
                for h in range(4):
                    c0 = h * 32
                    qa = qd[e, pl.ds(c0, 16)]
                    qb = qd[e, pl.ds(c0 + 16, 16)]
                    ka = kt[e, pl.ds(c0, 16)]
                    kb = kt[e, pl.ds(c0 + 16, 16)]
                    sc = jnp.sum(qa * ka + qb * kb)
                    ev = jnp.exp(jnp.full((16,), sc, _f32))
                    kt[e, pl.ds(c0, 16)] = vt[e, pl.ds(c0, 16)] * ev
                    kt[e, pl.ds(c0 + 16, 16)] = vt[e, pl.ds(c0 + 16, 16)] * ev
                    exrow = jnp.where(iota16 == h, ev, exrow)
                kt[e, pl.ds(128, 16)] = jnp.where(iota16 < 4, exrow,
                                                  jnp.zeros((16,), _f32))
                return carry2

            lax.fori_loop(0, B, edge_body, 0)

            pltpu.sync_copy(kt, acc_sh.at[dstv], add=True)
            return carry

        lax.fori_loop(0, NBLK, block, 0)
        plsc.subcore_barrier()

        def dump_chunk(off, sz):
            pltpu.sync_copy(acc_sh.at[pl.ds(off, sz)], kt.at[pl.ds(0, sz)])
            pltpu.sync_copy(kt.at[pl.ds(0, sz)], accc.at[pl.ds(off, sz)])

        @pl.when(s < 15)
        def _():
            for_chunks(NPT_MAIN, lambda o, n: dump_chunk(rbase + o, n))

        @pl.when(s == 15)
        def _():
            for_chunks(NPT_LAST, lambda o, n: dump_chunk(15 * NPT_MAIN + o, n))

    @pl.when(c == 0)
    def _():
        run(q0, k0, v0, acc0)

    @pl.when(c == 1)
    def _():
        run(q1, k1, v1, acc1)


def _dot(a, b):
    return jnp.dot(a, b, preferred_element_type=_f32)


def _qkvs_body0(x, wi, bi, wq, bq, wk, bk, wv, bv, ws, bs,
                q0, q1, k0, k1, v0, v1, skip):
    h = _dot(x[...], wi[...]) + bi[...]
    _qkvs_common(h, wq, bq, wk, bk, wv, bv, ws, bs,
                 q0, q1, k0, k1, v0, v1, skip)


def _qkvs_body(pre, s1, s2, gamma, beta, wq, bq, wk, bk, wv, bv, ws, bs,
               q0, q1, k0, k1, v0, v1, skip):
    mean = s1[...] * (1.0 / N)
    var = s2[...] * (1.0 / N) - mean * mean
    inv = lax.rsqrt(var + EPS)
    h = jnp.maximum((pre[...] - mean) * inv * gamma[...] + beta[...], 0.0)
    _qkvs_common(h, wq, bq, wk, bk, wv, bv, ws, bs,
                 q0, q1, k0, k1, v0, v1, skip)


def _qkvs_common(h, wq, bq, wk, bk, wv, bv, ws, bs,
                 q0, q1, k0, k1, v0, v1, skip):
    q = (_dot(h, wq[...]) + bq[...]) * INV_SQRT_C
    q0[...] = q[:, :128]
    q1[...] = q[:, 128:]
    k = _dot(h, wk[...]) + bk[...]
    zpad = jnp.zeros((k.shape[0], 16), _f32)
    k0[...] = jnp.concatenate([k[:, :128], zpad], axis=1)
    k1[...] = jnp.concatenate([k[:, 128:], zpad], axis=1)
    v = _dot(h, wv[...]) + bv[...]
    v0[...] = v[:, :128]
    v1[...] = v[:, 128:]
    skip[...] = _dot(h, ws[...]) + bs[...]


def _gate_body(skip, a0, a1, wb, pre, s1, s2):
    i = pl.program_id(0)
    sk = skip[...]
    acc0 = a0[...]
    acc1 = a1[...]
    den8 = jnp.concatenate([acc0[:, 128:132], acc1[:, 128:132]], axis=1)
    den_e = jnp.broadcast_to(den8[:, :, None], (RB, 8, 32)).reshape(RB, HC)
    ag = jnp.concatenate([acc0[:, :128], acc1[:, :128]], axis=1)
    ag = ag / (den_e + 1e-16)
    w_s = wb[0:1, :] + wb[2:3, :]
    w_a = wb[1:2, :] - wb[2:3, :]
    gl = (jnp.sum(sk * w_s, axis=1, keepdims=True)
          + jnp.sum(ag * w_a, axis=1, keepdims=True))
    g = jax.nn.sigmoid(gl)
    p = g * sk + (1.0 - g) * ag
    pre[...] = p

    @pl.when(i == 0)
    def _():
        s1[...] = jnp.zeros_like(s1)
        s2[...] = jnp.zeros_like(s2)

    s1[...] += jnp.sum(p, axis=0, keepdims=True)
    s2[...] += jnp.sum(p * p, axis=0, keepdims=True)


def _final_body(pre, s1, s2, gamma, beta, bat, wf, bf, out, acc, cnt):
    i = pl.program_id(0)
    mean = s1[...] * (1.0 / N)
    var = s2[...] * (1.0 / N) - mean * mean
    inv = lax.rsqrt(var + EPS)
    h = jnp.maximum((pre[...] - mean) * inv * gamma[...] + beta[...], 0.0)
    b = bat[...].reshape(1, RB)
    oh = (lax.broadcasted_iota(jnp.int32, (N_GRAPHS, RB), 0) == b).astype(_f32)

    @pl.when(i == 0)
    def _():
        acc[...] = jnp.zeros_like(acc)
        cnt[...] = jnp.zeros_like(cnt)

    acc[...] += lax.dot_general(oh, h, (((1,), (0,)), ((), ())),
                                preferred_element_type=_f32)
    cnt[...] += jnp.sum(oh, axis=1, keepdims=True)

    @pl.when(i == NRB - 1)
    def _():
        pooled = acc[...] / jnp.maximum(cnt[...], 1.0)
        out[...] = _dot(pooled, wf[...]) + bf[...]


def _row_spec(w):
    return pl.BlockSpec((RB, w), lambda i: (i, 0))


def _full_spec(shape):
    nd = len(shape)
    return pl.BlockSpec(shape, lambda i: (0,) * nd)


_QKVS_OUTS = (
    [jax.ShapeDtypeStruct((N, 128), _f32)] * 2
    + [jax.ShapeDtypeStruct((N, 144), _f32)] * 2
    + [jax.ShapeDtypeStruct((N, 128), _f32)] * 2
    + [jax.ShapeDtypeStruct((N, HC), _f32)]
)
_QKVS_OUT_SPECS = ([_row_spec(128)] * 2 + [_row_spec(144)] * 2
                   + [_row_spec(128)] * 2 + [_row_spec(HC)])


def _qkvs0_call(x, wi, bi, wq, bq, wk, bk, wv, bv, ws, bs):
    return pl.pallas_call(
        _qkvs_body0,
        grid=(NRB,),
        in_specs=[_row_spec(D_FEAT),
                  _full_spec((D_FEAT, HIDDEN)), _full_spec((1, HIDDEN))]
                 + [_full_spec((HIDDEN, HC)), _full_spec((1, HC))] * 4,
        out_specs=_QKVS_OUT_SPECS,
        out_shape=_QKVS_OUTS,
    )(x, wi, bi, wq, bq, wk, bk, wv, bv, ws, bs)


def _qkvs_call(pre, s1, s2, gamma, beta, wq, bq, wk, bk, wv, bv, ws, bs):
    return pl.pallas_call(
        _qkvs_body,
        grid=(NRB,),
        in_specs=[_row_spec(HC)] + [_full_spec((1, HC))] * 4
                 + [_full_spec((HC, HC)), _full_spec((1, HC))] * 4,
        out_specs=_QKVS_OUT_SPECS,
        out_shape=_QKVS_OUTS,
    )(pre, s1, s2, gamma, beta, wq, bq, wk, bk, wv, bv, ws, bs)


def _gate_call(skip, a0, a1, wb3):
    return pl.pallas_call(
        _gate_body,
        grid=(NRB,),
        in_specs=[_row_spec(HC), _row_spec(144), _row_spec(144),
                  _full_spec((3, HC))],
        out_specs=[_row_spec(HC), _full_spec((1, HC)), _full_spec((1, HC))],
        out_shape=[jax.ShapeDtypeStruct((N, HC), _f32),
                   jax.ShapeDtypeStruct((1, HC), _f32),
                   jax.ShapeDtypeStruct((1, HC), _f32)],
    )(skip, a0, a1, wb3)


def _final_call(pre, s1, s2, gamma, beta, b3, wf, bf):
    return pl.pallas_call(
        _final_body,
        grid=(NRB,),
        in_specs=[_row_spec(HC)] + [_full_spec((1, HC))] * 4
                 + [pl.BlockSpec((1, 1, RB), lambda i: (i, 0, 0)),
                    _full_spec((HC, OUT)), _full_spec((1, OUT))],
        out_specs=[_full_spec((N_GRAPHS, OUT))],
        out_shape=[jax.ShapeDtypeStruct((N_GRAPHS, OUT), _f32)],
        scratch_shapes=[pltpu.VMEM((N_GRAPHS, HC), _f32),
                        pltpu.VMEM((N_GRAPHS, 1), _f32)],
    )(pre, s1, s2, gamma, beta, b3, wf, bf)[0]


def kernel(x, edge_index, batch, params):
    src = edge_index[0]
    dst = edge_index[1]
    z144 = jnp.zeros((N, 144), _f32)
    r1 = lambda a: a.reshape(1, -1)

    layers = params['layers']
    lp = layers[0]
    q0, q1, k0, k1, v0, v1, skip = _qkvs0_call(
        x, params['W_init'], r1(params['b_init']),
        lp['Wq'], r1(lp['bq']), lp['Wk'], r1(lp['bk']),
        lp['Wv'], r1(lp['bv']), lp['Wskip'], r1(lp['bskip']))
    acc0, acc1 = _sc_attn(q0, q1, k0, k1, v0, v1, dst, src, z144)
    pre, s1, s2 = _gate_call(skip, acc0, acc1, lp['Wbeta'].reshape(3, HC))

    for li in range(1, 4):
        prev = layers[li - 1]
        lp = layers[li]
        q0, q1, k0, k1, v0, v1, skip = _qkvs_call(
            pre, s1, s2, r1(prev['bn_gamma']), r1(prev['bn_beta']),
            lp['Wq'], r1(lp['bq']), lp['Wk'], r1(lp['bk']),
            lp['Wv'], r1(lp['bv']), lp['Wskip'], r1(lp['bskip']))
        acc0, acc1 = _sc_attn(q0, q1, k0, k1, v0, v1, dst, src, z144)
        pre, s1, s2 = _gate_call(skip, acc0, acc1, lp['Wbeta'].reshape(3, HC))

    lp = layers[3]
    b3 = batch.reshape(NRB, 1, RB)
    return _final_call(pre, s1, s2, r1(lp['bn_gamma']), r1(lp['bn_beta']),
                       b3, params['W_final'], r1(params['b_final']))
